# baseline (speedup 1.0000x reference)
import functools

import jax
import jax.numpy as jnp
from jax import lax
from jax.experimental import pallas as pl
from jax.experimental.pallas import tpu as pltpu
from jax.experimental.pallas import tpu_sc as plsc

NV = 64
NC = 8
K = 16
MAX_ITER = 20
BISECT_STEPS = 10
TC_BISECT_STEPS = 7
BUDGET = 8.0
BT = 512
NW = 32
SC_BATCH = 512
GW = NV * 16


def _tc_body(s_ref, o_ref):
    blk = s_ref[...]
    b = blk.shape[0]
    t1 = jnp.transpose(blk).reshape(NC, 8, b)
    s = jnp.stack([t1[:, r, :] for r in range(8)], axis=0)

    def outer(_, carry):
        u_t, p_t = carry
        u_roll = jnp.roll(u_t, -1, axis=1)
        y = jnp.concatenate([u_t, u_roll], axis=0) + p_t

        x0 = jnp.clip(y, 0.0, 1.0)
        s0 = jnp.sum(x0, axis=0)
        need = s0 > BUDGET
        hi = jnp.maximum(jnp.max(y, axis=0), 1e-6)
        lo = jnp.zeros_like(hi)
        g_lo = s0
        g_hi = jnp.zeros_like(hi)

        for _ in range(TC_BISECT_STEPS):
            mid = 0.5 * (lo + hi)
            g = jnp.sum(jnp.clip(y - mid[None], 0.0, 1.0), axis=0)
            gt = g > BUDGET
            lo = jnp.where(gt, mid, lo)
            g_lo = jnp.where(gt, g, g_lo)
            hi = jnp.where(gt, hi, mid)
            g_hi = jnp.where(gt, g_hi, g)

        tau = lo + (hi - lo) * (g_lo - BUDGET) / jnp.maximum(
            g_lo - g_hi, 1e-12)
        tau = jnp.clip(tau, lo, hi)
        x1 = jnp.clip(y - tau[None], 0.0, 1.0)
        z = jnp.where(need[None], x1, x0)

        p_new = y - z
        z_hi = jnp.roll(z[8:], 1, axis=1)
        u_new = (z[:8] + z_hi) * 0.5
        return u_new, p_new

    u0 = s
    p0 = jnp.zeros((K, NC, b), jnp.float32)
    u_t, _ = jax.lax.fori_loop(0, MAX_ITER, outer, (u0, p0))
    v = jnp.concatenate([u_t[:, j, :] for j in range(NC)], axis=0)
    o_ref[...] = jnp.transpose(v)


def _tc_solve(scores):
    batch = scores.shape[0]
    return pl.pallas_call(
        _tc_body,
        grid=(batch // BT,),
        in_specs=[pl.BlockSpec((BT, NV), lambda i: (i, 0))],
        out_specs=pl.BlockSpec((BT, NV), lambda i: (i, 0)),
        out_shape=jax.ShapeDtypeStruct((batch, NV), jnp.float32),
    )(scores)


def _tree_sum(vals):
    vals = list(vals)
    while len(vals) > 1:
        vals = [a + b for a, b in zip(vals[0::2], vals[1::2])]
    return vals[0]


def _tree_max(vals):
    vals = list(vals)
    while len(vals) > 1:
        vals = [jnp.maximum(a, b) for a, b in zip(vals[0::2], vals[1::2])]
    return vals[0]


def _make_sc_solve(batch):
    ngroups = batch // 16
    ng = ngroups // NW

    @functools.partial(
        pl.kernel,
        mesh=plsc.VectorSubcoreMesh(core_axis_name="c", subcore_axis_name="s"),
        out_type=jax.ShapeDtypeStruct((ngroups * GW,), jnp.float32),
        scratch_types=[
            pltpu.VMEM((NV * 16,), jnp.float32),
            pltpu.VMEM((NC * K * 16,), jnp.float32),
            pltpu.VMEM((NC * K * 16,), jnp.float32),
        ],
    )
    def _sc_solve(s_hbm, out_hbm, u_v, p_v, z_v):
        wid = lax.axis_index("s") * 2 + lax.axis_index("c")

        def group_body(g, carry):
            base = (wid * ng + g) * GW
            pltpu.sync_copy(s_hbm.at[pl.ds(base, GW)], u_v)
            zero = jnp.zeros((16,), jnp.float32)
            for r in range(NC * K):
                p_v[pl.ds(r * 16, 16)] = zero

            def iter_body(_, c2):
                for c in range(NC):
                    y = [u_v[pl.ds(((8 * c + k) % NV) * 16, 16)]
                         + p_v[pl.ds((c * K + k) * 16, 16)]
                         for k in range(K)]
                    x0 = [jnp.clip(y[k], 0.0, 1.0) for k in range(K)]
                    need = _tree_sum(x0) > BUDGET
                    hi = jnp.maximum(_tree_max(y), 1e-6)
                    lo = jnp.zeros((16,), jnp.float32)

                    def bis(_, lohi):
                        lo_, hi_ = lohi
                        mid = 0.5 * (lo_ + hi_)
                        acc = _tree_sum(
                            [jnp.clip(y[k] - mid, 0.0, 1.0)
                             for k in range(K)])
                        gt = acc > BUDGET
                        return (jnp.where(gt, mid, lo_),
                                jnp.where(gt, hi_, mid))

                    lo, hi = lax.fori_loop(0, BISECT_STEPS, bis, (lo, hi))
                    tau = 0.5 * (lo + hi)
                    for k in range(K):
                        zk = jnp.where(need,
                                       jnp.clip(y[k] - tau, 0.0, 1.0),
                                       x0[k])
                        p_v[pl.ds((c * K + k) * 16, 16)] = y[k] - zk
                        z_v[pl.ds((c * K + k) * 16, 16)] = zk
                for j in range(NC):
                    for r in range(8):
                        u_v[pl.ds((8 * j + r) * 16, 16)] = 0.5 * (
                            z_v[pl.ds((K * j + r) * 16, 16)]
                            + z_v[pl.ds(
                                (K * ((j - 1) % NC) + 8 + r) * 16, 16)])
                return c2

            lax.fori_loop(0, MAX_ITER, iter_body, 0)
            pltpu.sync_copy(u_v, out_hbm.at[pl.ds(base, GW)])
            return carry

        lax.fori_loop(0, ng, group_body, 0)

    return _sc_solve


def _sc_solve_batch(scores):
    batch = scores.shape[0]
    ngroups = batch // 16
    blocks = jnp.transpose(scores.reshape(ngroups, 16, NV), (0, 2, 1))
    out = _make_sc_solve(batch)(blocks.reshape(-1))
    out3 = out.reshape(ngroups, NV, 16)
    return jnp.transpose(out3, (0, 2, 1)).reshape(batch, NV)


@jax.jit
def kernel(scores):
    s = scores.astype(jnp.float32)
    tc_part = _tc_solve(s[:-SC_BATCH])
    sc_part = _sc_solve_batch(s[-SC_BATCH:])
    return jnp.concatenate([tc_part, sc_part], axis=0)


# speedup vs baseline: 1.6484x; 1.6484x over previous
import functools

import jax
import jax.numpy as jnp
from jax import lax
from jax.experimental import pallas as pl
from jax.experimental.pallas import tpu as pltpu
from jax.experimental.pallas import tpu_sc as plsc

NV = 64
NC = 8
K = 16
MAX_ITER = 20
BISECT_STEPS = 10
TC_BISECT_STEPS = 7
BUDGET = 8.0
BT = 512
NW = 32
SC_BATCH = 512
GW = NV * 16


def _tc_body(s_ref, o_ref):
    s = s_ref[...]
    b = s.shape[-1]

    def outer(_, carry):
        u_t, p_t = carry
        u_roll = jnp.roll(u_t, -1, axis=1)
        y = jnp.concatenate([u_t, u_roll], axis=0) + p_t

        x0 = jnp.clip(y, 0.0, 1.0)
        s0 = jnp.sum(x0, axis=0)
        need = s0 > BUDGET
        hi = jnp.maximum(jnp.max(y, axis=0), 1e-6)
        lo = jnp.zeros_like(hi)
        g_lo = s0
        g_hi = jnp.zeros_like(hi)

        for _ in range(TC_BISECT_STEPS):
            mid = 0.5 * (lo + hi)
            g = jnp.sum(jnp.clip(y - mid[None], 0.0, 1.0), axis=0)
            gt = g > BUDGET
            lo = jnp.where(gt, mid, lo)
            g_lo = jnp.where(gt, g, g_lo)
            hi = jnp.where(gt, hi, mid)
            g_hi = jnp.where(gt, g_hi, g)

        tau = lo + (hi - lo) * (g_lo - BUDGET) / jnp.maximum(
            g_lo - g_hi, 1e-12)
        tau = jnp.clip(tau, lo, hi)
        x1 = jnp.clip(y - tau[None], 0.0, 1.0)
        z = jnp.where(need[None], x1, x0)

        p_new = y - z
        z_hi = jnp.roll(z[8:], 1, axis=1)
        u_new = (z[:8] + z_hi) * 0.5
        return u_new, p_new

    u0 = s
    p0 = jnp.zeros((K, NC, b), jnp.float32)
    u_t, _ = jax.lax.fori_loop(0, MAX_ITER, outer, (u0, p0))
    o_ref[...] = u_t


def _tc_solve(scores):
    batch = scores.shape[0]
    st = jnp.transpose(scores.reshape(batch, NC, 8), (2, 1, 0))
    out = pl.pallas_call(
        _tc_body,
        grid=(batch // BT,),
        in_specs=[pl.BlockSpec((8, NC, BT), lambda i: (0, 0, i))],
        out_specs=pl.BlockSpec((8, NC, BT), lambda i: (0, 0, i)),
        out_shape=jax.ShapeDtypeStruct((8, NC, batch), jnp.float32),
    )(st)
    return jnp.transpose(out, (2, 1, 0)).reshape(batch, NV)


def _tree_sum(vals):
    vals = list(vals)
    while len(vals) > 1:
        vals = [a + b for a, b in zip(vals[0::2], vals[1::2])]
    return vals[0]


def _tree_max(vals):
    vals = list(vals)
    while len(vals) > 1:
        vals = [jnp.maximum(a, b) for a, b in zip(vals[0::2], vals[1::2])]
    return vals[0]


def _make_sc_solve(batch):
    ngroups = batch // 16
    ng = ngroups // NW

    @functools.partial(
        pl.kernel,
        mesh=plsc.VectorSubcoreMesh(core_axis_name="c", subcore_axis_name="s"),
        out_type=jax.ShapeDtypeStruct((ngroups * GW,), jnp.float32),
        scratch_types=[
            pltpu.VMEM((NV * 16,), jnp.float32),
            pltpu.VMEM((NC * K * 16,), jnp.float32),
            pltpu.VMEM((NC * K * 16,), jnp.float32),
        ],
    )
    def _sc_solve(s_hbm, out_hbm, u_v, p_v, z_v):
        wid = lax.axis_index("s") * 2 + lax.axis_index("c")

        def group_body(g, carry):
            base = (wid * ng + g) * GW
            pltpu.sync_copy(s_hbm.at[pl.ds(base, GW)], u_v)
            zero = jnp.zeros((16,), jnp.float32)
            for r in range(NC * K):
                p_v[pl.ds(r * 16, 16)] = zero

            def iter_body(_, c2):
                for c in range(NC):
                    y = [u_v[pl.ds(((8 * c + k) % NV) * 16, 16)]
                         + p_v[pl.ds((c * K + k) * 16, 16)]
                         for k in range(K)]
                    x0 = [jnp.clip(y[k], 0.0, 1.0) for k in range(K)]
                    need = _tree_sum(x0) > BUDGET
                    hi = jnp.maximum(_tree_max(y), 1e-6)
                    lo = jnp.zeros((16,), jnp.float32)

                    def bis(_, lohi):
                        lo_, hi_ = lohi
                        mid = 0.5 * (lo_ + hi_)
                        acc = _tree_sum(
                            [jnp.clip(y[k] - mid, 0.0, 1.0)
                             for k in range(K)])
                        gt = acc > BUDGET
                        return (jnp.where(gt, mid, lo_),
                                jnp.where(gt, hi_, mid))

                    lo, hi = lax.fori_loop(0, BISECT_STEPS, bis, (lo, hi))
                    tau = 0.5 * (lo + hi)
                    for k in range(K):
                        zk = jnp.where(need,
                                       jnp.clip(y[k] - tau, 0.0, 1.0),
                                       x0[k])
                        p_v[pl.ds((c * K + k) * 16, 16)] = y[k] - zk
                        z_v[pl.ds((c * K + k) * 16, 16)] = zk
                for j in range(NC):
                    for r in range(8):
                        u_v[pl.ds((8 * j + r) * 16, 16)] = 0.5 * (
                            z_v[pl.ds((K * j + r) * 16, 16)]
                            + z_v[pl.ds(
                                (K * ((j - 1) % NC) + 8 + r) * 16, 16)])
                return c2

            lax.fori_loop(0, MAX_ITER, iter_body, 0)
            pltpu.sync_copy(u_v, out_hbm.at[pl.ds(base, GW)])
            return carry

        lax.fori_loop(0, ng, group_body, 0)

    return _sc_solve


def _sc_solve_batch(scores):
    batch = scores.shape[0]
    ngroups = batch // 16
    blocks = jnp.transpose(scores.reshape(ngroups, 16, NV), (0, 2, 1))
    out = _make_sc_solve(batch)(blocks.reshape(-1))
    out3 = out.reshape(ngroups, NV, 16)
    return jnp.transpose(out3, (0, 2, 1)).reshape(batch, NV)


@jax.jit
def kernel(scores):
    s = scores.astype(jnp.float32)
    tc_part = _tc_solve(s[:-SC_BATCH])
    sc_part = _sc_solve_batch(s[-SC_BATCH:])
    return jnp.concatenate([tc_part, sc_part], axis=0)


# speedup vs baseline: 1.7472x; 1.0600x over previous
import functools

import jax
import jax.numpy as jnp
from jax import lax
from jax.experimental import pallas as pl
from jax.experimental.pallas import tpu as pltpu
from jax.experimental.pallas import tpu_sc as plsc

NV = 64
NC = 8
K = 16
MAX_ITER = 20
BISECT_STEPS = 10
TC_BISECT_STEPS = 6
BUDGET = 8.0
BT = 512
NW = 32
SC_BATCH = 512
GW = NV * 16


def _tc_body(s_ref, o_ref):
    s = s_ref[...]
    b = s.shape[-1]

    def outer(_, carry):
        u_t, p_t = carry
        u_roll = jnp.roll(u_t, -1, axis=1)
        y = jnp.concatenate([u_t, u_roll], axis=0) + p_t

        x0 = jnp.clip(y, 0.0, 1.0)
        s0 = jnp.sum(x0, axis=0)
        need = s0 > BUDGET
        hi = jnp.maximum(jnp.max(y, axis=0), 1e-6)
        lo = jnp.zeros_like(hi)
        g_lo = s0
        g_hi = jnp.zeros_like(hi)

        for _ in range(TC_BISECT_STEPS):
            mid = 0.5 * (lo + hi)
            g = jnp.sum(jnp.clip(y - mid[None], 0.0, 1.0), axis=0)
            gt = g > BUDGET
            lo = jnp.where(gt, mid, lo)
            g_lo = jnp.where(gt, g, g_lo)
            hi = jnp.where(gt, hi, mid)
            g_hi = jnp.where(gt, g_hi, g)

        tau = lo + (hi - lo) * (g_lo - BUDGET) / jnp.maximum(
            g_lo - g_hi, 1e-12)
        tau = jnp.clip(tau, lo, hi)
        x1 = jnp.clip(y - tau[None], 0.0, 1.0)
        z = jnp.where(need[None], x1, x0)

        p_new = y - z
        z_hi = jnp.roll(z[8:], 1, axis=1)
        u_new = (z[:8] + z_hi) * 0.5
        return u_new, p_new

    u0 = s
    p0 = jnp.zeros((K, NC, b), jnp.float32)
    u_t, _ = jax.lax.fori_loop(0, MAX_ITER, outer, (u0, p0))
    o_ref[...] = u_t


def _tc_solve(scores):
    batch = scores.shape[0]
    st = jnp.transpose(scores.reshape(batch, NC, 8), (2, 1, 0))
    out = pl.pallas_call(
        _tc_body,
        grid=(batch // BT,),
        in_specs=[pl.BlockSpec((8, NC, BT), lambda i: (0, 0, i))],
        out_specs=pl.BlockSpec((8, NC, BT), lambda i: (0, 0, i)),
        out_shape=jax.ShapeDtypeStruct((8, NC, batch), jnp.float32),
    )(st)
    return jnp.transpose(out, (2, 1, 0)).reshape(batch, NV)


def _tree_sum(vals):
    vals = list(vals)
    while len(vals) > 1:
        vals = [a + b for a, b in zip(vals[0::2], vals[1::2])]
    return vals[0]


def _tree_max(vals):
    vals = list(vals)
    while len(vals) > 1:
        vals = [jnp.maximum(a, b) for a, b in zip(vals[0::2], vals[1::2])]
    return vals[0]


def _make_sc_solve(batch):
    ngroups = batch // 16
    ng = ngroups // NW

    @functools.partial(
        pl.kernel,
        mesh=plsc.VectorSubcoreMesh(core_axis_name="c", subcore_axis_name="s"),
        out_type=jax.ShapeDtypeStruct((ngroups * GW,), jnp.float32),
        scratch_types=[
            pltpu.VMEM((NV * 16,), jnp.float32),
            pltpu.VMEM((NC * K * 16,), jnp.float32),
            pltpu.VMEM((NC * K * 16,), jnp.float32),
        ],
    )
    def _sc_solve(s_hbm, out_hbm, u_v, p_v, z_v):
        wid = lax.axis_index("s") * 2 + lax.axis_index("c")

        def group_body(g, carry):
            base = (wid * ng + g) * GW
            pltpu.sync_copy(s_hbm.at[pl.ds(base, GW)], u_v)
            zero = jnp.zeros((16,), jnp.float32)
            for r in range(NC * K):
                p_v[pl.ds(r * 16, 16)] = zero

            def iter_body(_, c2):
                for c in range(NC):
                    y = [u_v[pl.ds(((8 * c + k) % NV) * 16, 16)]
                         + p_v[pl.ds((c * K + k) * 16, 16)]
                         for k in range(K)]
                    x0 = [jnp.clip(y[k], 0.0, 1.0) for k in range(K)]
                    need = _tree_sum(x0) > BUDGET
                    hi = jnp.maximum(_tree_max(y), 1e-6)
                    lo = jnp.zeros((16,), jnp.float32)

                    def bis(_, lohi):
                        lo_, hi_ = lohi
                        mid = 0.5 * (lo_ + hi_)
                        acc = _tree_sum(
                            [jnp.clip(y[k] - mid, 0.0, 1.0)
                             for k in range(K)])
                        gt = acc > BUDGET
                        return (jnp.where(gt, mid, lo_),
                                jnp.where(gt, hi_, mid))

                    lo, hi = lax.fori_loop(0, BISECT_STEPS, bis, (lo, hi))
                    tau = 0.5 * (lo + hi)
                    for k in range(K):
                        zk = jnp.where(need,
                                       jnp.clip(y[k] - tau, 0.0, 1.0),
                                       x0[k])
                        p_v[pl.ds((c * K + k) * 16, 16)] = y[k] - zk
                        z_v[pl.ds((c * K + k) * 16, 16)] = zk
                for j in range(NC):
                    for r in range(8):
                        u_v[pl.ds((8 * j + r) * 16, 16)] = 0.5 * (
                            z_v[pl.ds((K * j + r) * 16, 16)]
                            + z_v[pl.ds(
                                (K * ((j - 1) % NC) + 8 + r) * 16, 16)])
                return c2

            lax.fori_loop(0, MAX_ITER, iter_body, 0)
            pltpu.sync_copy(u_v, out_hbm.at[pl.ds(base, GW)])
            return carry

        lax.fori_loop(0, ng, group_body, 0)

    return _sc_solve


def _sc_solve_batch(scores):
    batch = scores.shape[0]
    ngroups = batch // 16
    blocks = jnp.transpose(scores.reshape(ngroups, 16, NV), (0, 2, 1))
    out = _make_sc_solve(batch)(blocks.reshape(-1))
    out3 = out.reshape(ngroups, NV, 16)
    return jnp.transpose(out3, (0, 2, 1)).reshape(batch, NV)


@jax.jit
def kernel(scores):
    s = scores.astype(jnp.float32)
    tc_part = _tc_solve(s[:-SC_BATCH])
    sc_part = _sc_solve_batch(s[-SC_BATCH:])
    return jnp.concatenate([tc_part, sc_part], axis=0)
